# Initial kernel scaffold; baseline (speedup 1.0000x reference)
import jax
import jax.numpy as jnp
from jax.experimental import pallas as pl


def kernel(x, edge_index, W1, a_src1, a_dst1, b1, W2, a_src2, a_dst2, b2):
    raise NotImplementedError("write your pallas kernel here")


# speedup vs baseline: 1.0814x; 1.0814x over previous
import functools

import jax
import jax.numpy as jnp
from jax.experimental import pallas as pl

N = 10000
E = 320000
F_IN = 128
HID = 64
HEADS = 8

NP = 10240


def _mm_kernel(x_ref, w_ref, as_ref, ad_ref, h_ref, s_ref, d_ref):
    h = jnp.dot(x_ref[...], w_ref[...], preferred_element_type=jnp.float32)
    h_ref[...] = h
    hh = h.reshape(h.shape[0], HEADS, HID)
    s_ref[...] = (hh * as_ref[...][None]).sum(-1)
    d_ref[...] = (hh * ad_ref[...][None]).sum(-1)


def _dense1(x, W1, a_src1, a_dst1):
    xp = jnp.zeros((NP, F_IN), x.dtype).at[:N].set(x)
    BLK = 512
    h, als, ald = pl.pallas_call(
        _mm_kernel,
        grid=(NP // BLK,),
        in_specs=[
            pl.BlockSpec((BLK, F_IN), lambda i: (i, 0)),
            pl.BlockSpec((F_IN, HEADS * HID), lambda i: (0, 0)),
            pl.BlockSpec((HEADS, HID), lambda i: (0, 0)),
            pl.BlockSpec((HEADS, HID), lambda i: (0, 0)),
        ],
        out_specs=[
            pl.BlockSpec((BLK, HEADS * HID), lambda i: (i, 0)),
            pl.BlockSpec((BLK, HEADS), lambda i: (i, 0)),
            pl.BlockSpec((BLK, HEADS), lambda i: (i, 0)),
        ],
        out_shape=[
            jax.ShapeDtypeStruct((NP, HEADS * HID), jnp.float32),
            jax.ShapeDtypeStruct((NP, HEADS), jnp.float32),
            jax.ShapeDtypeStruct((NP, HEADS), jnp.float32),
        ],
    )(xp, W1, a_src1, a_dst1)
    return h[:N], als[:N], ald[:N]


def _mm2_kernel(h_ref, w_ref, b1_ref, o_ref):
    t = h_ref[...] + b1_ref[...]
    h = jnp.where(t > 0, t, jnp.exp(jnp.minimum(t, 0.0)) - 1.0)
    o_ref[...] = jnp.dot(h, w_ref[...], preferred_element_type=jnp.float32)


def _dense2(agg, b1, W2):
    W2p = jnp.zeros((HEADS * HID, 128), jnp.float32).at[:, :1].set(W2)
    aggp = jnp.zeros((NP, HEADS * HID), jnp.float32).at[:N].set(agg)
    BLK = 512
    out = pl.pallas_call(
        _mm2_kernel,
        grid=(NP // BLK,),
        in_specs=[
            pl.BlockSpec((BLK, HEADS * HID), lambda i: (i, 0)),
            pl.BlockSpec((HEADS * HID, 128), lambda i: (0, 0)),
            pl.BlockSpec((1, HEADS * HID), lambda i: (0, 0)),
        ],
        out_specs=pl.BlockSpec((BLK, 128), lambda i: (i, 0)),
        out_shape=jax.ShapeDtypeStruct((NP, 128), jnp.float32),
    )(aggp, W2p, b1.reshape(1, HEADS * HID))
    return out[:N, :1]


def _edge_phase(h, als, ald, src, dst, heads, n_nodes):
    loop = jnp.arange(n_nodes, dtype=src.dtype)
    src_f = jnp.concatenate([src, loop])
    dst_f = jnp.concatenate([dst, loop])
    e = als[src_f] + ald[dst_f]
    e = jnp.where(e >= 0, e, 0.2 * e)
    emax = jax.ops.segment_max(e, dst_f, num_segments=n_nodes)
    ex = jnp.exp(e - emax[dst_f])
    denom = jax.ops.segment_sum(ex, dst_f, num_segments=n_nodes)
    hh = h.reshape(n_nodes, heads, -1)
    msg = hh[src_f] * ex[:, :, None]
    agg = jax.ops.segment_sum(msg, dst_f, num_segments=n_nodes)
    agg = agg / (denom[:, :, None] + 1e-16)
    return agg


def kernel(x, edge_index, W1, a_src1, a_dst1, b1, W2, a_src2, a_dst2, b2):
    src = edge_index[0]
    dst = edge_index[1]
    h1, als1, ald1 = _dense1(x, W1, a_src1, a_dst1)
    agg1 = _edge_phase(h1, als1, ald1, src, dst, HEADS, N).reshape(N, HEADS * HID)
    h2 = _dense2(agg1, b1, W2)
    als2 = h2 * a_src2[0, 0]
    ald2 = h2 * a_dst2[0, 0]
    agg2 = _edge_phase(h2, als2, ald2, src, dst, 1, N)
    return agg2[:, 0, :] + b2
